# baseline (speedup 1.0000x reference)
import functools

import jax
import jax.numpy as jnp
from jax.experimental import pallas as pl

_CMID = 16


def _pconv_block_kernel(rel_ref, nbv_ref, w1_ref, b1_ref, w2_ref, b2_ref,
                        wl_ref, bl_ref, o_ref, *, blk, k, cin, cout):
    rel = rel_ref[...]
    h = jnp.maximum(rel @ w1_ref[...] + b1_ref[...], 0.0)
    h = h @ w2_ref[...] + b2_ref[...]
    nbv = nbv_ref[...].reshape(blk, k, cin)
    h3 = h.reshape(blk, k, _CMID)
    agg = jnp.einsum('bkc,bkm->bcm', nbv, h3,
                     preferred_element_type=jnp.float32) * (1.0 / k)
    out = agg.reshape(blk, cin * _CMID) @ wl_ref[...] + bl_ref[...]
    o_ref[...] = out


def _run_pconv(rel, nbv, p, blk=128):
    B, N, k, d = rel.shape
    cin = nbv.shape[-1]
    cout = p['Wl'].shape[-1]
    flat = rel.reshape(-1, d)
    ntot = flat.shape[0]
    mr = jnp.mean(flat, 0)
    cov = flat.T @ flat / ntot - jnp.outer(mr, mr)
    m = mr @ p['W1'] + p['b1']
    v = jnp.einsum('aj,ab,bj->j', p['W1'], cov, p['W1'])
    s = p['g1'] / jnp.sqrt(v + 1e-5)
    w1f = p['W1'] * s[None, :]
    b1f = (p['b1'] - m) * s + p['be1']

    bn = B * N
    rel2 = rel.reshape(bn * k, d)
    nbv2 = nbv.reshape(bn * k, cin)
    grid = (bn // blk,)
    out = pl.pallas_call(
        functools.partial(_pconv_block_kernel, blk=blk, k=k, cin=cin,
                          cout=cout),
        grid=grid,
        in_specs=[
            pl.BlockSpec((blk * k, d), lambda i: (i, 0)),
            pl.BlockSpec((blk * k, cin), lambda i: (i, 0)),
            pl.BlockSpec((d, _CMID), lambda i: (0, 0)),
            pl.BlockSpec((1, _CMID), lambda i: (0, 0)),
            pl.BlockSpec((_CMID, _CMID), lambda i: (0, 0)),
            pl.BlockSpec((1, _CMID), lambda i: (0, 0)),
            pl.BlockSpec((cin * _CMID, cout), lambda i: (0, 0)),
            pl.BlockSpec((1, cout), lambda i: (0, 0)),
        ],
        out_specs=pl.BlockSpec((blk, cout), lambda i: (i, 0)),
        out_shape=jax.ShapeDtypeStruct((bn, cout), jnp.float32),
    )(rel2, nbv2, w1f, b1f.reshape(1, _CMID), p['W2'],
      p['b2'].reshape(1, _CMID), p['Wl'], p['bl'].reshape(1, cout))
    return out.reshape(B, N, cout)


def kernel(ctx_coords, ctx_values, tgt_coords, params):
    B, C, d = ctx_coords.shape
    T = tgt_coords.shape[1]
    y = ctx_values.shape[-1]
    coords = jnp.concatenate([ctx_coords, tgt_coords], axis=1)
    N = C + T

    sq = jnp.sum(coords * coords, -1)
    d2 = sq[:, :, None] + sq[:, None, :] - 2.0 * jnp.einsum(
        'bnd,bmd->bnm', coords, coords)
    _, idx300 = jax.lax.top_k(-d2, 300)
    idx5 = idx300[:, :, :5]

    gat = jax.vmap(lambda arr, i: arr[i])
    nbc300 = gat(coords, idx300)
    rel300 = nbc300 - coords[:, :, None, :]
    nbc5 = nbc300[:, :, :5, :]
    rel5 = nbc5 - coords[:, :, None, :]

    density = jnp.concatenate(
        [jnp.ones_like(ctx_values), jnp.zeros((B, T, y), jnp.float32)], axis=1)
    signal = jnp.concatenate(
        [ctx_values, jnp.zeros((B, T, y), jnp.float32)], axis=1)

    dprime = _run_pconv(rel300, gat(density, idx300), params['ct'], blk=8)
    sprime = _run_pconv(rel300, gat(signal, idx300), params['ct'], blk=8)
    h = jnp.concatenate([dprime, sprime], axis=-1)
    h = jax.nn.relu(_run_pconv(rel5, gat(h, idx5), params['c1']))
    h = jax.nn.relu(_run_pconv(rel5, gat(h, idx5), params['c2']))
    h = jax.nn.relu(_run_pconv(rel5, gat(h, idx5), params['c3']))
    f = _run_pconv(rel5, gat(h, idx5), params['c4'])
    f_mu = f[:, C:, 0]
    f_sigma = jax.nn.softplus(f[:, C:, 1])
    sigma = jnp.eye(T, dtype=jnp.float32)[None, :, :] * f_sigma[:, :, None]
    return (f_mu, sigma)
